# Initial kernel scaffold; baseline (speedup 1.0000x reference)
import jax
import jax.numpy as jnp
from jax.experimental import pallas as pl


def kernel(state, task_inference_latent, mem_keys, mem_vals, params):
    raise NotImplementedError("write your pallas kernel here")


# speedup vs baseline: 4.9041x; 4.9041x over previous
import jax
import jax.numpy as jnp
from jax.experimental import pallas as pl

_B = 1024
_BB = 8
_H = 8
_K = 256
_E = 200
_V = 256
_TOPK = 32
_ROWS = _BB * _H


def _dnd_body(state_ref, lat_ref, keys_ref, vals_ref,
              wse, bse, wc1, bc1, wc2, bc2, wq, bq,
              wagg, bagg, wk1, bk1, wk2, bk2, wv1, bv1, wv2, bv2,
              mk_ref, mv_ref):
    f32 = jnp.float32

    def dot(a, b):
        return jax.lax.dot_general(a, b, (((1,), (0,)), ((), ())),
                                   preferred_element_type=f32)

    s = dot(state_ref[...], wse[...]) + bse[...]
    qc = jnp.concatenate([s, lat_ref[...]], axis=1)
    qc = dot(qc, wc1[...]) + bc1[...]
    qc = dot(qc, wc2[...]) + bc2[...]
    q_heads = [dot(qc, wq[:, _K * h:_K * (h + 1)]) + bq[:, _K * h:_K * (h + 1)]
               for h in range(_H)]

    sims = []
    for i in range(_BB):
        q_b = jnp.concatenate([q_heads[h][i:i + 1, :] for h in range(_H)],
                              axis=0)
        keys_b = keys_ref[:, i, :]
        sim_b = jax.lax.dot_general(q_b, keys_b, (((1,), (1,)), ((), ())),
                                    preferred_element_type=f32)
        sims.append(sim_b * (1.0 / 16.0))
    sim = jnp.concatenate(sims, axis=0)

    ub = jax.lax.bitcast_convert_type(sim, jnp.uint32)
    u = jnp.where(ub >= jnp.uint32(0x80000000), ~ub,
                  ub | jnp.uint32(0x80000000))
    prefix = jnp.zeros((_ROWS, 1), jnp.uint32)
    for bit in range(31, -1, -1):
        cand = prefix | jnp.uint32(1 << bit)
        cnt = jnp.sum((u >= cand).astype(jnp.int32), axis=1, keepdims=True)
        prefix = jnp.where(cnt >= _TOPK, cand, prefix)
    thr = prefix

    gt = u > thr
    cnt_gt = jnp.sum(gt.astype(jnp.int32), axis=1, keepdims=True)
    need = (_TOPK - cnt_gt).astype(f32)
    eq = u == thr
    ri = jax.lax.broadcasted_iota(jnp.int32, (_E, _E), 0)
    ci = jax.lax.broadcasted_iota(jnp.int32, (_E, _E), 1)
    ltri = (ri < ci).astype(f32)
    excl = dot(eq.astype(f32), ltri)
    mask = gt | (eq & (excl < need))

    m = jnp.max(sim, axis=1, keepdims=True)
    p = jnp.where(mask, jnp.exp(sim - m), 0.0)
    w = p / jnp.sum(p, axis=1, keepdims=True)

    r_list = []
    for i in range(_BB):
        w_b = w[_H * i:_H * i + _H, :]
        r_list.append(dot(w_b, vals_ref[:, i, :]))

    acc = bagg[...]
    for h in range(_H):
        rows_h = jnp.concatenate([r_list[i][h:h + 1, :] for i in range(_BB)],
                                 axis=0)
        acc = acc + dot(rows_h, wagg[_V * h:_V * (h + 1), :])

    mk = dot(dot(acc, wk1[...]) + bk1[...], wk2[...]) + bk2[...]
    mv = dot(dot(acc, wv1[...]) + bv1[...], wv2[...]) + bv2[...]
    mk_ref[...] = mk
    mv_ref[...] = mv


def kernel(state, task_inference_latent, mem_keys, mem_vals, params):
    f32 = jnp.float32
    wse = params["state_encoder"][0]["w"]
    c1, c2 = params["concat_query_encoder"]
    qe = params["query_encoder"][0]
    agg = params["value_aggregator"][0]
    k1, k2 = params["read_memory_to_key"]
    v1, v2 = params["read_memory_to_value"]

    def bb(b):
        return jnp.broadcast_to(b.astype(f32), (_BB, b.shape[0]))

    grid = _B // _BB
    full = lambda a: pl.BlockSpec(a.shape, lambda i: (0,) * a.ndim)
    in_specs = [
        pl.BlockSpec((_BB, state.shape[1]), lambda i: (i, 0)),
        pl.BlockSpec((_BB, task_inference_latent.shape[1]), lambda i: (i, 0)),
        pl.BlockSpec((_E, _BB, _K), lambda i: (0, i, 0)),
        pl.BlockSpec((_E, _BB, _V), lambda i: (0, i, 0)),
    ]
    weights = []
    for wmat, bvec in ((wse, params["state_encoder"][0]["b"]),
                       (c1["w"], c1["b"]), (c2["w"], c2["b"]),
                       (qe["w"], qe["b"]), (agg["w"], agg["b"]),
                       (k1["w"], k1["b"]), (k2["w"], k2["b"]),
                       (v1["w"], v1["b"]), (v2["w"], v2["b"])):
        weights.append(wmat.astype(f32))
        weights.append(bb(bvec))
    in_specs += [full(a) for a in weights]

    out_shape = (jax.ShapeDtypeStruct((_B, k2["w"].shape[1]), f32),
                 jax.ShapeDtypeStruct((_B, v2["w"].shape[1]), f32))
    out_specs = (pl.BlockSpec((_BB, k2["w"].shape[1]), lambda i: (i, 0)),
                 pl.BlockSpec((_BB, v2["w"].shape[1]), lambda i: (i, 0)))

    mk, mv = pl.pallas_call(
        _dnd_body,
        grid=(grid,),
        in_specs=in_specs,
        out_specs=out_specs,
        out_shape=out_shape,
    )(state, task_inference_latent, mem_keys, mem_vals, *weights)
    return mk, mv
